# baseline (speedup 1.0000x reference)
import jax
import jax.numpy as jnp
from jax import lax
from jax.experimental import pallas as pl
from jax.experimental.pallas import tpu as pltpu
from jax.experimental.pallas import tpu_sc as plsc

B, N, M, K = 8, 32768, 8192, 128
NC, NS, L = 2, 16, 16
NW = NC * NS
QPT = (B * N) // NW
GU = 8
CHUNKS = QPT // (GU * L)


def _nn_body(verts_hbm, coll_hbm, cv_hbm, out_hbm,
             vbuf, collbuf, cvbuf, sxbuf, sybuf, szbuf, obuf):
    wid = lax.axis_index("s") * NC + lax.axis_index("c")
    b = wid // (N // QPT)
    qoff = wid * QPT

    pltpu.sync_copy(cv_hbm, cvbuf)
    pltpu.sync_copy(verts_hbm.at[pl.ds(qoff * 3, QPT * 3)], vbuf)
    pltpu.sync_copy(coll_hbm.at[pl.ds(b * (M * 3), M * 3)], collbuf)

    for j in range(K // L):
        idx3 = cvbuf[pl.ds(j * L, L)] * 3
        sxbuf[pl.ds(j * L, L)] = plsc.load_gather(collbuf, [idx3])
        sybuf[pl.ds(j * L, L)] = plsc.load_gather(collbuf, [idx3 + 1])
        szbuf[pl.ds(j * L, L)] = plsc.load_gather(collbuf, [idx3 + 2])

    lanes = lax.iota(jnp.int32, L)
    bvec = jnp.full((L,), b, jnp.int32)
    inf = jnp.full((L,), jnp.inf, jnp.float32)
    zero = jnp.zeros((L,), jnp.int32)

    def chunk_body(c, carry):
        base = c * (GU * L)
        vx, vy, vz = [], [], []
        for g in range(GU):
            a3 = (base + g * L + lanes) * 3
            vx.append(plsc.load_gather(vbuf, [a3]))
            vy.append(plsc.load_gather(vbuf, [a3 + 1]))
            vz.append(plsc.load_gather(vbuf, [a3 + 2]))

        def k_body(k, bc):
            best, bidx = bc
            kv = jnp.full((L,), k, jnp.int32)
            sx = plsc.load_gather(sxbuf, [kv])
            sy = plsc.load_gather(sybuf, [kv])
            sz = plsc.load_gather(szbuf, [kv])
            nbest, nbidx = [], []
            for g in range(GU):
                dx = vx[g] - sx
                dy = vy[g] - sy
                dz = vz[g] - sz
                d2 = (dx * dx + dy * dy) + dz * dz
                m = d2 < best[g]
                nbest.append(jnp.where(m, d2, best[g]))
                nbidx.append(jnp.where(m, kv, bidx[g]))
            return nbest, nbidx

        best, bidx = lax.fori_loop(0, K, k_body, ([inf] * GU, [zero] * GU))
        for g in range(GU):
            q2 = (base + g * L + lanes) * 2
            plsc.store_scatter(obuf, [q2], bvec)
            plsc.store_scatter(obuf, [q2 + 1], bidx[g])
        return carry

    lax.fori_loop(0, CHUNKS, chunk_body, 0)
    pltpu.sync_copy(obuf, out_hbm.at[pl.ds(qoff * 2, QPT * 2)])


def kernel(vertices, collider, collision_vertices):
    mesh = plsc.VectorSubcoreMesh(core_axis_name="c", subcore_axis_name="s")
    run = pl.kernel(
        _nn_body,
        out_type=jax.ShapeDtypeStruct((B * N * 2,), jnp.int32),
        mesh=mesh,
        compiler_params=pltpu.CompilerParams(needs_layout_passes=False),
        scratch_types=[
            pltpu.VMEM((QPT * 3,), jnp.float32),
            pltpu.VMEM((M * 3,), jnp.float32),
            pltpu.VMEM((K,), jnp.int32),
            pltpu.VMEM((K,), jnp.float32),
            pltpu.VMEM((K,), jnp.float32),
            pltpu.VMEM((K,), jnp.float32),
            pltpu.VMEM((QPT * 2,), jnp.int32),
        ],
    )
    out = run(vertices.reshape(-1), collider.reshape(-1), collision_vertices)
    return out.reshape(B, N, 2)
